# baseline (speedup 1.0000x reference)
import functools

import numpy as np

import jax
import jax.numpy as jnp
from jax import lax
from jax.experimental import pallas as pl
from jax.experimental.pallas import tpu as pltpu
from jax.experimental.pallas import tpu_sc as plsc

NF = 30
FD = 1000
V = NF * FD
D = 32
NPAIR = NF * (NF - 1) // 2
GP = 4
NG = (NPAIR + GP - 1) // GP
KP = NG * GP * D
K = NPAIR * D
B = 1024
HB = B // 2
HT = HB // 8
NRT = B // 8
NW = 32
CH = 128
NCHUNK = B // CH

_PAIR_I = np.array([i for i in range(NF - 1) for _ in range(i + 1, NF)],
                   dtype=np.int32)
_PAIR_J = np.array([j for i in range(NF - 1) for j in range(i + 1, NF)],
                   dtype=np.int32)


def _sc_body(ftab_hbm, idxa_hbm, idxb_hbm, xiT_hbm, lin_hbm, cross_out,
             psum_out, psumsq_out, linpart_out, idx_a, idx_b, rows_a, rows_b,
             prod, stat_s, stat_q, lin_rows, sem_a, sem_b, sem_l):
    wid = lax.axis_index("s") * 2 + lax.axis_index("c")

    f_eff = jnp.minimum(wid, NF - 1)
    pltpu.sync_copy(xiT_hbm.at[f_eff], idx_a.at[0])
    cps = [
        pltpu.async_copy(lin_hbm.at[idx_a.at[0, c]],
                         lin_rows.at[pl.ds(c * CH, CH)], sem_l)
        for c in range(NCHUNK)
    ]
    for cp in cps:
        cp.wait()
    pltpu.sync_copy(
        lin_rows, linpart_out.at[pl.ds(pl.multiple_of(wid * B, CH), B)])

    nt_w = (NG - 1 - wid) // NW + 1
    NU = 16
    RU = 2 * CH
    sems = (sem_a, sem_b)

    def group_body(t, carry):
        g = wid + t * NW

        z = jnp.zeros((16,), jnp.float32)
        for q16 in range(8):
            stat_s[pl.ds(q16 * 16, 16)] = z
            stat_q[pl.ds(q16 * 16, 16)] = z

        def issue(n):
            s, u = divmod(n, 2)
            h, q = divmod(s, 4)
            par = n % 2
            ring = s % 2
            if u == 0:
                p = jnp.minimum(g * GP + q, NPAIR - 1)
                pltpu.sync_copy(idxa_hbm.at[p], idx_a.at[ring])
                pltpu.sync_copy(idxb_hbm.at[p], idx_b.at[ring])
            hs = []
            for cc in range(2):
                c = 4 * h + 2 * u + cc
                hs.append(pltpu.async_copy(
                    ftab_hbm.at[idx_a.at[ring, c]],
                    rows_a.at[par, pl.ds(cc * CH, CH), :], sems[par]))
                hs.append(pltpu.async_copy(
                    ftab_hbm.at[idx_b.at[ring, c]],
                    rows_b.at[par, pl.ds(cc * CH, CH), :], sems[par]))
            return hs

        def compute(n):
            s, u = divmod(n, 2)
            h, q = divmod(s, 4)
            par = n % 2

            def prow(rt2, st):
                s0, s1, q0, q1 = st
                for r8 in range(8):
                    rr = rt2 * 8 + r8
                    a0 = rows_a[par, rr, pl.ds(0, 16)]
                    a1 = rows_a[par, rr, pl.ds(16, 16)]
                    b0 = rows_b[par, rr, pl.ds(0, 16)]
                    b1 = rows_b[par, rr, pl.ds(16, 16)]
                    p0 = a0 * b0
                    p1 = a1 * b1
                    rt = u * (RU // 8) + rt2
                    prod[rt, r8, pl.ds(q * D, 16)] = p0
                    prod[rt, r8, pl.ds(q * D + 16, 16)] = p1
                    s0 = s0 + p0
                    s1 = s1 + p1
                    q0 = q0 + p0 * p0
                    q1 = q1 + p1 * p1
                return (s0, s1, q0, q1)

            zz = jnp.zeros((16,), jnp.float32)
            s0, s1, q0, q1 = lax.fori_loop(0, RU // 8, prow,
                                           (zz, zz, zz, zz))
            stat_s[pl.ds(q * D, 16)] = stat_s[pl.ds(q * D, 16)] + s0
            stat_s[pl.ds(q * D + 16, 16)] = (
                stat_s[pl.ds(q * D + 16, 16)] + s1)
            stat_q[pl.ds(q * D, 16)] = stat_q[pl.ds(q * D, 16)] + q0
            stat_q[pl.ds(q * D + 16, 16)] = (
                stat_q[pl.ds(q * D + 16, 16)] + q1)

        handles = {0: issue(0)}
        for n in range(NU):
            if n + 1 < NU:
                handles[(n + 1) % 2] = issue(n + 1)
            for cp in handles[n % 2]:
                cp.wait()
            compute(n)
            if n % 8 == 7:
                h = n // 8
                pltpu.sync_copy(prod, cross_out.at[pl.ds(h * HT, HT), g])

        off = pl.multiple_of(g * 128, 128)
        pltpu.sync_copy(stat_s, psum_out.at[pl.ds(off, 128)])
        pltpu.sync_copy(stat_q, psumsq_out.at[pl.ds(off, 128)])
        return carry

    lax.fori_loop(0, nt_w, group_body, 0)


_sc_gather = functools.partial(
    pl.kernel,
    out_type=[
        jax.ShapeDtypeStruct((NRT, NG, 8, 128), jnp.float32),
        jax.ShapeDtypeStruct((KP,), jnp.float32),
        jax.ShapeDtypeStruct((KP,), jnp.float32),
        jax.ShapeDtypeStruct((NW * B,), jnp.float32),
    ],
    mesh=plsc.VectorSubcoreMesh(core_axis_name="c", subcore_axis_name="s"),
    scratch_types=[
        pltpu.VMEM((2, NCHUNK, CH), jnp.int32),
        pltpu.VMEM((2, NCHUNK, CH), jnp.int32),
        pltpu.VMEM((2, 2 * CH, D), jnp.float32),
        pltpu.VMEM((2, 2 * CH, D), jnp.float32),
        pltpu.VMEM((HT, 8, GP * D), jnp.float32),
        pltpu.VMEM((GP * D,), jnp.float32),
        pltpu.VMEM((GP * D,), jnp.float32),
        pltpu.VMEM((B,), jnp.float32),
        pltpu.SemaphoreType.DMA,
        pltpu.SemaphoreType.DMA,
        pltpu.SemaphoreType.DMA,
    ],
    compiler_params=pltpu.CompilerParams(use_tc_tiling_on_sc=False),
)(_sc_body)


BC = 128
BT = BC // 8
GB = B // BC


def _tc_body(cross_ref, psum_ref, psq_ref, bn0g_ref, bn0b_ref, w1_ref, b1_ref,
             bn1g_ref, bn1b_ref, w2_ref, b2_ref, bn2g_ref, bn2b_ref,
             w3_ref, b3_ref, linT_ref, linb_ref, out_ref, h1_acc):
    g = pl.program_id(0)
    inv_b = 1.0 / B
    m = psum_ref[...] * inv_b
    var = psq_ref[...] * inv_b - m * m
    s = bn0g_ref[...] * lax.rsqrt(var + 1e-5)
    x4 = cross_ref[...]
    blk = jnp.swapaxes(x4, 1, 2).reshape(BC, KP)
    scaled = (blk - m) * s + bn0b_ref[...]
    h1_acc[pl.ds(g * BC, BC), :] = (
        lax.dot_general(scaled[:, :K], w1_ref[...],
                        (((1,), (1,)), ((), ())),
                        preferred_element_type=jnp.float32)
        + b1_ref[...])

    @pl.when(g == GB - 1)
    def _():
        h1 = h1_acc[...]
        m1 = jnp.mean(h1, axis=0, keepdims=True)
        v1 = jnp.mean((h1 - m1) ** 2, axis=0, keepdims=True)
        h = jnp.maximum(
            (h1 - m1) * lax.rsqrt(v1 + 1e-5) * bn1g_ref[...] + bn1b_ref[...],
            0.0)
        h2 = (jnp.dot(h, w2_ref[...], preferred_element_type=jnp.float32)
              + b2_ref[...])
        m2 = jnp.mean(h2, axis=0, keepdims=True)
        v2 = jnp.mean((h2 - m2) ** 2, axis=0, keepdims=True)
        h2 = jnp.maximum(
            (h2 - m2) * lax.rsqrt(v2 + 1e-5) * bn2g_ref[...] + bn2b_ref[...],
            0.0)
        h3 = jnp.sum(h2 * w3_ref[...], axis=1, keepdims=True) + b3_ref[...]
        linear = jnp.sum(linT_ref[...], axis=1, keepdims=True) + linb_ref[...]
        out_ref[...] = jax.nn.sigmoid(linear + h3)


def _tc_mlp(cross, psum, psumsq, bn0_g, bn0_b, W1, b1, bn1_g, bn1_b,
            W2, b2, bn2_g, bn2_b, W3, b3, linT, lin_bias):
    row = lambda a, n: a.reshape(1, n)
    return pl.pallas_call(
        _tc_body,
        grid=(GB,),
        in_specs=[
            pl.BlockSpec((BT, NG, 8, 128), lambda g: (g, 0, 0, 0)),
            pl.BlockSpec((1, KP), lambda g: (0, 0)),
            pl.BlockSpec((1, KP), lambda g: (0, 0)),
            pl.BlockSpec((1, KP), lambda g: (0, 0)),
            pl.BlockSpec((1, KP), lambda g: (0, 0)),
            pl.BlockSpec((400, K), lambda g: (0, 0)),
            pl.BlockSpec((1, 400), lambda g: (0, 0)),
            pl.BlockSpec((1, 400), lambda g: (0, 0)),
            pl.BlockSpec((1, 400), lambda g: (0, 0)),
            pl.BlockSpec((400, 400), lambda g: (0, 0)),
            pl.BlockSpec((1, 400), lambda g: (0, 0)),
            pl.BlockSpec((1, 400), lambda g: (0, 0)),
            pl.BlockSpec((1, 400), lambda g: (0, 0)),
            pl.BlockSpec((1, 400), lambda g: (0, 0)),
            pl.BlockSpec((1, 1), lambda g: (0, 0)),
            pl.BlockSpec((B, NF), lambda g: (0, 0)),
            pl.BlockSpec((1, 1), lambda g: (0, 0)),
        ],
        out_specs=pl.BlockSpec((B, 1), lambda g: (0, 0)),
        out_shape=jax.ShapeDtypeStruct((B, 1), jnp.float32),
        scratch_shapes=[pltpu.VMEM((B, 400), jnp.float32)],
        compiler_params=pltpu.CompilerParams(
            dimension_semantics=("arbitrary",)),
    )(cross, row(psum, KP), row(psumsq, KP), row(bn0_g, KP), row(bn0_b, KP),
      W1, row(b1, 400), row(bn1_g, 400), row(bn1_b, 400), W2, row(b2, 400),
      row(bn2_g, 400), row(bn2_b, 400), row(W3, 400), b3.reshape(1, 1),
      linT, lin_bias.reshape(1, 1))


def kernel(x, additional, lin_table, lin_bias, ffm_tables, bn0_g, bn0_b,
           W1, b1, bn1_g, bn1_b, W2, b2, bn2_g, bn2_b, W3, b3):
    del additional
    xs = jnp.concatenate(
        [x[:, :3], x[:, 4:8], x[:, 10:15], x[:, 17:19], x[:, 21:24],
         x[:, 26:]], axis=1).astype(jnp.int32)
    offsets = jnp.arange(NF, dtype=jnp.int32) * FD
    xi = xs + offsets[None, :]

    xiTf = xi.T
    rep_i = jnp.concatenate(
        [jnp.broadcast_to(xiTf[i:i + 1], (NF - 1 - i, B))
         for i in range(NF - 1)], axis=0)
    rep_j = jnp.concatenate(
        [xiTf[i + 1:] for i in range(NF - 1)], axis=0)
    pi_col = jnp.asarray(_PAIR_I)[:, None]
    pj_col = jnp.asarray(_PAIR_J)[:, None]
    idxa = (rep_i + pj_col * V).reshape(NPAIR, NCHUNK, CH)
    idxb = (rep_j + pi_col * V).reshape(NPAIR, NCHUNK, CH)

    ftab = ffm_tables.reshape(NF * V, D)
    lin_flat = lin_table.reshape(V)
    xiT = xiTf.reshape(NF, NCHUNK, CH)

    cross, psum, psumsq, linpart = _sc_gather(ftab, idxa, idxb, xiT, lin_flat)

    padk = lambda a: jnp.pad(a, (0, KP - K))
    out = _tc_mlp(cross, psum, psumsq, padk(bn0_g), padk(bn0_b),
                  W1.T, b1, bn1_g, bn1_b, W2, b2, bn2_g, bn2_b, W3, b3,
                  linpart.reshape(NW, B)[:NF].T, lin_bias)
    return out[:, 0]


# speedup vs baseline: 1.0608x; 1.0608x over previous
import functools

import jax
import jax.numpy as jnp
from jax import lax
from jax.experimental import pallas as pl
from jax.experimental.pallas import tpu as pltpu
from jax.experimental.pallas import tpu_sc as plsc

NF = 30
FD = 1000
V = NF * FD
D = 32
NPAIR = NF * (NF - 1) // 2
GP = 4
NG = (NPAIR + GP - 1) // GP
KP = NG * GP * D
K = NPAIR * D
B = 1024
HB = B // 2
HT = HB // 8
NRT = B // 8
NW = 32
CH = 128
NCHUNK = B // CH

TB = 1000
NBT = V // TB
NUNIT = NF * NBT
NT_FMT = (NUNIT + NW - 1) // NW


def _fmt_body(ffmv_hbm, ftab_out, xin, yout, lsem, wsem):
    wid = lax.axis_index("s") * 2 + lax.axis_index("c")

    def unit_fv(t):
        u = jnp.minimum(wid + t * NW, NUNIT - 1)
        f = u // NBT
        v0 = (u - f * NBT) * TB
        return f, v0

    def issue_load(t):
        f, v0 = unit_fv(t)
        return pltpu.async_copy(
            ffmv_hbm.at[f, :, pl.ds(v0, TB)],
            xin.at[pl.ds((t % 2) * D, D), :], lsem)

    loads = {0: issue_load(0)}
    writes = {}
    for t in range(NT_FMT):
        par = t % 2
        if t + 1 < NT_FMT:
            loads[(t + 1) % 2] = issue_load(t + 1)
        loads[par].wait()
        if t - 2 in writes:
            writes.pop(t - 2).wait()

        def rowloop(v, c, par=par):
            iota_lo = lax.iota(jnp.int32, 16)
            vs = jnp.zeros((16,), jnp.int32) + v
            g0 = plsc.load_gather(xin, [iota_lo + par * D, vs])
            g1 = plsc.load_gather(xin, [iota_lo + (par * D + 16), vs])
            yout[par, v, pl.ds(0, 16)] = g0
            yout[par, v, pl.ds(16, 16)] = g1
            return c

        lax.fori_loop(0, TB, rowloop, 0)
        f, v0 = unit_fv(t)
        writes[t] = pltpu.async_copy(
            yout.at[par], ftab_out.at[f, pl.ds(v0, TB), :], wsem)
    for w in writes.values():
        w.wait()


_sc_format = functools.partial(
    pl.kernel,
    out_type=[jax.ShapeDtypeStruct((NF, V, D), jnp.float32)],
    mesh=plsc.VectorSubcoreMesh(core_axis_name="c", subcore_axis_name="s"),
    scratch_types=[
        pltpu.VMEM((2 * D, TB), jnp.float32),
        pltpu.VMEM((2, TB, D), jnp.float32),
        pltpu.SemaphoreType.DMA,
        pltpu.SemaphoreType.DMA,
    ],
    compiler_params=pltpu.CompilerParams(use_tc_tiling_on_sc=False,
                                         needs_layout_passes=False),
)(_fmt_body)


def _unrank(p):
    i = jnp.int32(0)
    for u in range(1, NF):
        i = i + (p >= (u * (2 * NF - 1 - u)) // 2).astype(jnp.int32)
    j = p - (i * (2 * NF - 1 - i)) // 2 + i + 1
    return i, j


def _sc_body(ftab_hbm, xiT_hbm, lin_hbm, cross_out,
             psum_out, psumsq_out, linpart_out, idx_a, idx_b, rows_a, rows_b,
             prod, stat_s, stat_q, lin_rows, sem_a, sem_b, sem_l):
    wid = lax.axis_index("s") * 2 + lax.axis_index("c")

    f_eff = jnp.minimum(wid, NF - 1)
    pltpu.sync_copy(xiT_hbm.at[f_eff], idx_a.at[0])
    cps = [
        pltpu.async_copy(lin_hbm.at[idx_a.at[0, c]],
                         lin_rows.at[pl.ds(c * CH, CH)], sem_l)
        for c in range(NCHUNK)
    ]
    for cp in cps:
        cp.wait()
    pltpu.sync_copy(
        lin_rows, linpart_out.at[pl.ds(pl.multiple_of(wid * B, CH), B)])

    nt_w = (NG - 1 - wid) // NW + 1
    NU = 16
    RU = 2 * CH
    sems = (sem_a, sem_b)

    def group_body(t, carry):
        g = wid + t * NW

        z = jnp.zeros((16,), jnp.float32)
        for q16 in range(8):
            stat_s[pl.ds(q16 * 16, 16)] = z
            stat_q[pl.ds(q16 * 16, 16)] = z

        def issue(n):
            s, u = divmod(n, 2)
            h, q = divmod(s, 4)
            par = n % 2
            ring = s % 2
            p = jnp.minimum(g * GP + q, NPAIR - 1)
            i, j = _unrank(p)
            if u == 0:
                pltpu.sync_copy(xiT_hbm.at[i], idx_a.at[ring])
                pltpu.sync_copy(xiT_hbm.at[j], idx_b.at[ring])
            hs = []
            for cc in range(2):
                c = 4 * h + 2 * u + cc
                hs.append(pltpu.async_copy(
                    ftab_hbm.at[j].at[idx_a.at[ring, c]],
                    rows_a.at[par, pl.ds(cc * CH, CH), :], sems[par]))
                hs.append(pltpu.async_copy(
                    ftab_hbm.at[i].at[idx_b.at[ring, c]],
                    rows_b.at[par, pl.ds(cc * CH, CH), :], sems[par]))
            return hs

        def compute(n):
            s, u = divmod(n, 2)
            h, q = divmod(s, 4)
            par = n % 2

            def prow(rt2, st):
                s0, s1, q0, q1 = st
                for r8 in range(8):
                    rr = rt2 * 8 + r8
                    a0 = rows_a[par, rr, pl.ds(0, 16)]
                    a1 = rows_a[par, rr, pl.ds(16, 16)]
                    b0 = rows_b[par, rr, pl.ds(0, 16)]
                    b1 = rows_b[par, rr, pl.ds(16, 16)]
                    p0 = a0 * b0
                    p1 = a1 * b1
                    rt = u * (RU // 8) + rt2
                    prod[rt, r8, pl.ds(q * D, 16)] = p0
                    prod[rt, r8, pl.ds(q * D + 16, 16)] = p1
                    s0 = s0 + p0
                    s1 = s1 + p1
                    q0 = q0 + p0 * p0
                    q1 = q1 + p1 * p1
                return (s0, s1, q0, q1)

            zz = jnp.zeros((16,), jnp.float32)
            s0, s1, q0, q1 = lax.fori_loop(0, RU // 8, prow,
                                           (zz, zz, zz, zz))
            stat_s[pl.ds(q * D, 16)] = stat_s[pl.ds(q * D, 16)] + s0
            stat_s[pl.ds(q * D + 16, 16)] = (
                stat_s[pl.ds(q * D + 16, 16)] + s1)
            stat_q[pl.ds(q * D, 16)] = stat_q[pl.ds(q * D, 16)] + q0
            stat_q[pl.ds(q * D + 16, 16)] = (
                stat_q[pl.ds(q * D + 16, 16)] + q1)

        handles = {0: issue(0)}
        for n in range(NU):
            if n + 1 < NU:
                handles[(n + 1) % 2] = issue(n + 1)
            for cp in handles[n % 2]:
                cp.wait()
            compute(n)
            if n % 8 == 7:
                h = n // 8
                pltpu.sync_copy(prod, cross_out.at[pl.ds(h * HT, HT), g])

        off = pl.multiple_of(g * 128, 128)
        pltpu.sync_copy(stat_s, psum_out.at[pl.ds(off, 128)])
        pltpu.sync_copy(stat_q, psumsq_out.at[pl.ds(off, 128)])
        return carry

    lax.fori_loop(0, nt_w, group_body, 0)


_sc_gather = functools.partial(
    pl.kernel,
    out_type=[
        jax.ShapeDtypeStruct((NRT, NG, 8, 128), jnp.float32),
        jax.ShapeDtypeStruct((KP,), jnp.float32),
        jax.ShapeDtypeStruct((KP,), jnp.float32),
        jax.ShapeDtypeStruct((NW * B,), jnp.float32),
    ],
    mesh=plsc.VectorSubcoreMesh(core_axis_name="c", subcore_axis_name="s"),
    scratch_types=[
        pltpu.VMEM((2, NCHUNK, CH), jnp.int32),
        pltpu.VMEM((2, NCHUNK, CH), jnp.int32),
        pltpu.VMEM((2, 2 * CH, D), jnp.float32),
        pltpu.VMEM((2, 2 * CH, D), jnp.float32),
        pltpu.VMEM((HT, 8, GP * D), jnp.float32),
        pltpu.VMEM((GP * D,), jnp.float32),
        pltpu.VMEM((GP * D,), jnp.float32),
        pltpu.VMEM((B,), jnp.float32),
        pltpu.SemaphoreType.DMA,
        pltpu.SemaphoreType.DMA,
        pltpu.SemaphoreType.DMA,
    ],
    compiler_params=pltpu.CompilerParams(use_tc_tiling_on_sc=False),
)(_sc_body)


BC = 128
BT = BC // 8
GB = B // BC


def _tc_body(cross_ref, psum_ref, psq_ref, bn0g_ref, bn0b_ref, w1_ref, b1_ref,
             bn1g_ref, bn1b_ref, w2_ref, b2_ref, bn2g_ref, bn2b_ref,
             w3_ref, b3_ref, linT_ref, linb_ref, out_ref, h1_acc):
    g = pl.program_id(0)
    inv_b = 1.0 / B
    m = psum_ref[...] * inv_b
    var = psq_ref[...] * inv_b - m * m
    s = bn0g_ref[...] * lax.rsqrt(var + 1e-5)
    x4 = cross_ref[...]
    blk = jnp.swapaxes(x4, 1, 2).reshape(BC, KP)
    scaled = (blk - m) * s + bn0b_ref[...]
    h1_acc[pl.ds(g * BC, BC), :] = (
        lax.dot_general(scaled[:, :K], w1_ref[...],
                        (((1,), (1,)), ((), ())),
                        preferred_element_type=jnp.float32)
        + b1_ref[...])

    @pl.when(g == GB - 1)
    def _():
        h1 = h1_acc[...]
        m1 = jnp.mean(h1, axis=0, keepdims=True)
        v1 = jnp.mean((h1 - m1) ** 2, axis=0, keepdims=True)
        h = jnp.maximum(
            (h1 - m1) * lax.rsqrt(v1 + 1e-5) * bn1g_ref[...] + bn1b_ref[...],
            0.0)
        h2 = (jnp.dot(h, w2_ref[...], preferred_element_type=jnp.float32)
              + b2_ref[...])
        m2 = jnp.mean(h2, axis=0, keepdims=True)
        v2 = jnp.mean((h2 - m2) ** 2, axis=0, keepdims=True)
        h2 = jnp.maximum(
            (h2 - m2) * lax.rsqrt(v2 + 1e-5) * bn2g_ref[...] + bn2b_ref[...],
            0.0)
        h3 = jnp.sum(h2 * w3_ref[...], axis=1, keepdims=True) + b3_ref[...]
        linear = jnp.sum(linT_ref[...], axis=1, keepdims=True) + linb_ref[...]
        out_ref[...] = jax.nn.sigmoid(linear + h3)


def _tc_mlp(cross, psum, psumsq, bn0_g, bn0_b, W1, b1, bn1_g, bn1_b,
            W2, b2, bn2_g, bn2_b, W3, b3, linT, lin_bias):
    row = lambda a, n: a.reshape(1, n)
    return pl.pallas_call(
        _tc_body,
        grid=(GB,),
        in_specs=[
            pl.BlockSpec((BT, NG, 8, 128), lambda g: (g, 0, 0, 0)),
            pl.BlockSpec((1, KP), lambda g: (0, 0)),
            pl.BlockSpec((1, KP), lambda g: (0, 0)),
            pl.BlockSpec((1, KP), lambda g: (0, 0)),
            pl.BlockSpec((1, KP), lambda g: (0, 0)),
            pl.BlockSpec((400, K), lambda g: (0, 0)),
            pl.BlockSpec((1, 400), lambda g: (0, 0)),
            pl.BlockSpec((1, 400), lambda g: (0, 0)),
            pl.BlockSpec((1, 400), lambda g: (0, 0)),
            pl.BlockSpec((400, 400), lambda g: (0, 0)),
            pl.BlockSpec((1, 400), lambda g: (0, 0)),
            pl.BlockSpec((1, 400), lambda g: (0, 0)),
            pl.BlockSpec((1, 400), lambda g: (0, 0)),
            pl.BlockSpec((1, 400), lambda g: (0, 0)),
            pl.BlockSpec((1, 1), lambda g: (0, 0)),
            pl.BlockSpec((B, NF), lambda g: (0, 0)),
            pl.BlockSpec((1, 1), lambda g: (0, 0)),
        ],
        out_specs=pl.BlockSpec((B, 1), lambda g: (0, 0)),
        out_shape=jax.ShapeDtypeStruct((B, 1), jnp.float32),
        scratch_shapes=[pltpu.VMEM((B, 400), jnp.float32)],
        compiler_params=pltpu.CompilerParams(
            dimension_semantics=("arbitrary",)),
    )(cross, row(psum, KP), row(psumsq, KP), row(bn0_g, KP), row(bn0_b, KP),
      W1, row(b1, 400), row(bn1_g, 400), row(bn1_b, 400), W2, row(b2, 400),
      row(bn2_g, 400), row(bn2_b, 400), row(W3, 400), b3.reshape(1, 1),
      linT, lin_bias.reshape(1, 1))


def kernel(x, additional, lin_table, lin_bias, ffm_tables, bn0_g, bn0_b,
           W1, b1, bn1_g, bn1_b, W2, b2, bn2_g, bn2_b, W3, b3):
    del additional
    xs = jnp.concatenate(
        [x[:, :3], x[:, 4:8], x[:, 10:15], x[:, 17:19], x[:, 21:24],
         x[:, 26:]], axis=1).astype(jnp.int32)
    offsets = jnp.arange(NF, dtype=jnp.int32) * FD
    xi = xs + offsets[None, :]

    lin_flat = lin_table.reshape(V)
    xiT = xi.T.reshape(NF, NCHUNK, CH)

    ffm_v = jnp.transpose(ffm_tables, (0, 2, 1))
    (ftab3,) = _sc_format(ffm_v)
    cross, psum, psumsq, linpart = _sc_gather(ftab3, xiT, lin_flat)

    padk = lambda a: jnp.pad(a, (0, KP - K))
    out = _tc_mlp(cross, psum, psumsq, padk(bn0_g), padk(bn0_b),
                  W1.T, b1, bn1_g, bn1_b, W2, b2, bn2_g, bn2_b, W3, b3,
                  linpart.reshape(NW, B)[:NF].T, lin_bias)
    return out[:, 0]
